# baseline (speedup 1.0000x reference)
import functools

import jax
import jax.numpy as jnp
from jax import lax
from jax.experimental import pallas as pl
from jax.experimental.pallas import tpu as pltpu
from jax.experimental.pallas import tpu_sc as plsc

_D = 64
_B = 4096
_QL = 20
_DL = 200
_TOTAL = _B * (_QL + _DL)
_NC = 2
_NS = 16
_NW = _NC * _NS
_PER_W = _TOTAL // _NW
_CHUNK = 128
_NCHUNK = _PER_W // _CHUNK
_NSLOT = 5
_LAG = 3

_mesh = plsc.VectorSubcoreMesh(core_axis_name="c", subcore_axis_name="s")


@functools.partial(
    pl.kernel,
    mesh=_mesh,
    out_type=jax.ShapeDtypeStruct((_TOTAL, _D), jnp.float32),
    scratch_types=[
        pltpu.VMEM((_PER_W,), jnp.int32),
        pltpu.VMEM((_NSLOT, _CHUNK, 2 * _D), jnp.float32),
    ]
    + [pltpu.SemaphoreType.DMA] * (2 * _NSLOT),
    compiler_params=pltpu.CompilerParams(use_tc_tiling_on_sc=False),
)
def _gather_all(idx_hbm, table_hbm, out_hbm, idx_all, rows, *sems):
    gsem = sems[:_NSLOT]
    osem = sems[_NSLOT:]
    wid = lax.axis_index("s") * _NC + lax.axis_index("c")
    base = wid * _PER_W

    pltpu.sync_copy(idx_hbm.at[pl.ds(base, _PER_W)], idx_all)

    def start_gather(c, b):
        isl = idx_all.at[pl.ds(pl.multiple_of(c * _CHUNK, _CHUNK), _CHUNK)]
        pltpu.async_copy(table_hbm.at[isl], rows.at[b], gsem[b])

    def wait_gather(b):
        pltpu.make_async_copy(
            table_hbm.at[idx_all.at[pl.ds(0, _CHUNK)]], rows.at[b], gsem[b]
        ).wait()

    def fixup(c, b):
        cbase = pl.multiple_of(c * _CHUNK, _CHUNK)
        minv = idx_all[pl.ds(cbase, 16)]
        for k in range(1, _CHUNK // 16):
            minv = jnp.minimum(minv, idx_all[pl.ds(cbase + k * 16, 16)])

        m = minv[0]
        for lane in range(1, 16):
            m = jnp.minimum(m, minv[lane])

        @pl.when(m <= 0)
        def _():
            def slow_group(g, carry):
                tokv = idx_all[pl.ds(cbase + g * 16, 16)]
                for lane in range(16):
                    @pl.when(tokv[lane] <= 0)
                    def _zr():
                        r = g * 16 + lane
                        for k in range(_D // 16):
                            rows.at[b][r, pl.ds(k * 16, 16)] = jnp.zeros(
                                (16,), jnp.float32)
                return carry

            lax.fori_loop(0, _CHUNK // 16, slow_group, 0)

    def start_write(c, b):
        off = pl.multiple_of(base + c * _CHUNK, _CHUNK)
        pltpu.async_copy(
            rows.at[b].at[pl.ds(0, _CHUNK), pl.ds(0, _D)],
            out_hbm.at[pl.ds(off, _CHUNK)],
            osem[b],
        )

    def wait_write(b):
        pltpu.make_async_copy(
            rows.at[b].at[pl.ds(0, _CHUNK), pl.ds(0, _D)],
            out_hbm.at[pl.ds(base, _CHUNK)],
            osem[b],
        ).wait()

    for b in range(_LAG):
        start_gather(b, b)
    for c in range(_LAG, _NSLOT):
        start_gather(c, c)
        wait_gather(c - _LAG)
        start_write(c - _LAG, c - _LAG)

    def group(g, carry):
        for b in range(_NSLOT):
            c = g * _NSLOT + b
            wait_write(b)
            start_gather(c, b)
            bj = (b + _NSLOT - _LAG) % _NSLOT
            wait_gather(bj)
            start_write(c - _LAG, bj)
        return carry

    lax.fori_loop(1, _NCHUNK // _NSLOT, group, 0, unroll=False)

    for j in range(_NCHUNK - _LAG, _NCHUNK):
        bj = j % _NSLOT
        wait_gather(bj)
        start_write(j, bj)
    for b in range(_NSLOT):
        wait_write(b)


def kernel(query_tokens, document_tokens, embedding_table):
    q = query_tokens.astype(jnp.int32)
    d = document_tokens.astype(jnp.int32)
    idx = jnp.concatenate([q, d], axis=1).reshape(_TOTAL)
    ids = jnp.arange(100000, dtype=jnp.int32)
    tbl = jnp.where((ids > 0)[:, None], embedding_table, 0.0)
    table128 = jnp.pad(tbl, ((0, 0), (0, _D)))
    out = _gather_all(idx, table128)
    return out.reshape(_B, _QL + _DL, _D)


# speedup vs baseline: 1.1615x; 1.1615x over previous
import functools

import jax
import jax.numpy as jnp
from jax import lax
from jax.experimental import pallas as pl
from jax.experimental.pallas import tpu as pltpu
from jax.experimental.pallas import tpu_sc as plsc

_D = 64
_B = 4096
_QL = 20
_DL = 200
_TOTAL = _B * (_QL + _DL)
_NC = 2
_NS = 16
_NW = _NC * _NS
_PER_W = _TOTAL // _NW
_CHUNK = 128
_NCHUNK = _PER_W // _CHUNK
_NSLOT = 10
_LAG = 5

_mesh = plsc.VectorSubcoreMesh(core_axis_name="c", subcore_axis_name="s")


@functools.partial(
    pl.kernel,
    mesh=_mesh,
    out_type=jax.ShapeDtypeStruct((_TOTAL, _D), jnp.float32),
    scratch_types=[
        pltpu.VMEM((_PER_W,), jnp.int32),
        pltpu.VMEM((_NSLOT, _CHUNK, _D), jnp.float32),
    ]
    + [pltpu.SemaphoreType.DMA] * (2 * _NSLOT),
    compiler_params=pltpu.CompilerParams(use_tc_tiling_on_sc=False),
)
def _gather_all(idx_hbm, table_hbm, out_hbm, idx_all, rows, *sems):
    gsem = sems[:_NSLOT]
    osem = sems[_NSLOT:]
    wid = lax.axis_index("s") * _NC + lax.axis_index("c")
    base = wid * _PER_W

    pltpu.sync_copy(idx_hbm.at[pl.ds(base, _PER_W)], idx_all)

    def start_gather(c, b):
        isl = idx_all.at[pl.ds(pl.multiple_of(c * _CHUNK, _CHUNK), _CHUNK)]
        pltpu.async_copy(table_hbm.at[isl], rows.at[b], gsem[b])

    def wait_gather(b):
        pltpu.make_async_copy(
            table_hbm.at[idx_all.at[pl.ds(0, _CHUNK)]], rows.at[b], gsem[b]
        ).wait()

    def fixup(c, b):
        cbase = pl.multiple_of(c * _CHUNK, _CHUNK)
        minv = idx_all[pl.ds(cbase, 16)]
        for k in range(1, _CHUNK // 16):
            minv = jnp.minimum(minv, idx_all[pl.ds(cbase + k * 16, 16)])

        m = minv[0]
        for lane in range(1, 16):
            m = jnp.minimum(m, minv[lane])

        @pl.when(m <= 0)
        def _():
            def slow_group(g, carry):
                tokv = idx_all[pl.ds(cbase + g * 16, 16)]
                for lane in range(16):
                    @pl.when(tokv[lane] <= 0)
                    def _zr():
                        r = g * 16 + lane
                        for k in range(_D // 16):
                            rows.at[b][r, pl.ds(k * 16, 16)] = jnp.zeros(
                                (16,), jnp.float32)
                return carry

            lax.fori_loop(0, _CHUNK // 16, slow_group, 0)

    def start_write(c, b):
        off = pl.multiple_of(base + c * _CHUNK, _CHUNK)
        pltpu.async_copy(rows.at[b], out_hbm.at[pl.ds(off, _CHUNK)], osem[b])

    def wait_write(b):
        pltpu.make_async_copy(
            rows.at[b], out_hbm.at[pl.ds(base, _CHUNK)], osem[b]
        ).wait()

    for b in range(_LAG):
        start_gather(b, b)
    for c in range(_LAG, _NSLOT):
        start_gather(c, c)
        wait_gather(c - _LAG)
        fixup(c - _LAG, c - _LAG)
        start_write(c - _LAG, c - _LAG)

    def group(g, carry):
        for b in range(_NSLOT):
            c = g * _NSLOT + b
            wait_write(b)
            start_gather(c, b)
            bj = (b + _NSLOT - _LAG) % _NSLOT
            wait_gather(bj)
            fixup(c - _LAG, bj)
            start_write(c - _LAG, bj)
        return carry

    lax.fori_loop(1, _NCHUNK // _NSLOT, group, 0, unroll=False)

    for j in range(_NCHUNK - _LAG, _NCHUNK):
        bj = j % _NSLOT
        wait_gather(bj)
        fixup(j, bj)
        start_write(j, bj)
    for b in range(_NSLOT):
        wait_write(b)


def kernel(query_tokens, document_tokens, embedding_table):
    q = query_tokens.astype(jnp.int32)
    d = document_tokens.astype(jnp.int32)
    idx = jnp.concatenate([q, d], axis=1).reshape(_TOTAL)
    out = _gather_all(idx, embedding_table)
    return out.reshape(_B, _QL + _DL, _D)


# speedup vs baseline: 1.1705x; 1.0078x over previous
import functools

import jax
import jax.numpy as jnp
from jax import lax
from jax.experimental import pallas as pl
from jax.experimental.pallas import tpu as pltpu
from jax.experimental.pallas import tpu_sc as plsc

_D = 64
_B = 4096
_QL = 20
_DL = 200
_TOTAL = _B * (_QL + _DL)
_NC = 2
_NS = 16
_NW = _NC * _NS
_PER_W = _TOTAL // _NW
_CHUNK = 128
_NCHUNK = _PER_W // _CHUNK
_NSLOT = 4
_LAG = 2

_mesh = plsc.VectorSubcoreMesh(core_axis_name="c", subcore_axis_name="s")


@functools.partial(
    pl.kernel,
    mesh=_mesh,
    out_type=jax.ShapeDtypeStruct((_TOTAL, _D), jnp.float32),
    scratch_types=[
        pltpu.VMEM((_PER_W,), jnp.int32),
        pltpu.VMEM((_NSLOT, _CHUNK, _D), jnp.float32),
    ]
    + [pltpu.SemaphoreType.DMA] * (2 * _NSLOT),
    compiler_params=pltpu.CompilerParams(use_tc_tiling_on_sc=False),
)
def _gather_all(idx_hbm, table_hbm, out_hbm, idx_all, rows, *sems):
    gsem = sems[:_NSLOT]
    osem = sems[_NSLOT:]
    wid = lax.axis_index("s") * _NC + lax.axis_index("c")
    base = wid * _PER_W

    pltpu.sync_copy(idx_hbm.at[pl.ds(base, _PER_W)], idx_all)

    def start_gather(c, b):
        isl = idx_all.at[pl.ds(pl.multiple_of(c * _CHUNK, _CHUNK), _CHUNK)]
        pltpu.async_copy(table_hbm.at[isl], rows.at[b], gsem[b])

    def wait_gather(c, b):
        isl = idx_all.at[pl.ds(pl.multiple_of(c * _CHUNK, _CHUNK), _CHUNK)]
        pltpu.make_async_copy(table_hbm.at[isl], rows.at[b], gsem[b]).wait()

    def start_write(c, b):
        off = pl.multiple_of(base + c * _CHUNK, _CHUNK)
        pltpu.async_copy(rows.at[b], out_hbm.at[pl.ds(off, _CHUNK)], osem[b])

    def wait_write(c, b):
        off = pl.multiple_of(base + c * _CHUNK, _CHUNK)
        pltpu.make_async_copy(
            rows.at[b], out_hbm.at[pl.ds(off, _CHUNK)], osem[b]
        ).wait()

    for b in range(_LAG):
        start_gather(b, b)
    for c in range(_LAG, _NSLOT):
        start_gather(c, c)
        wait_gather(c - _LAG, c - _LAG)
        start_write(c - _LAG, c - _LAG)

    def group(g, carry):
        for b in range(_NSLOT):
            c = g * _NSLOT + b
            wait_write(c - _NSLOT, b)
            start_gather(c, b)
            bj = (b + _NSLOT - _LAG) % _NSLOT
            wait_gather(c - _LAG, bj)
            start_write(c - _LAG, bj)
        return carry

    lax.fori_loop(1, _NCHUNK // _NSLOT, group, 0, unroll=False)

    for j in range(_NCHUNK - _LAG, _NCHUNK):
        bj = j % _NSLOT
        wait_gather(j, bj)
        start_write(j, bj)
    for b in range(_NSLOT):
        wait_write(_NCHUNK - _NSLOT + b, b)


def kernel(query_tokens, document_tokens, embedding_table):
    q = query_tokens.astype(jnp.int32)
    d = document_tokens.astype(jnp.int32)
    idx = jnp.concatenate([q, d], axis=1).reshape(_TOTAL)
    table = embedding_table.at[0].set(0.0)
    out = _gather_all(idx, table)
    return out.reshape(_B, _QL + _DL, _D)
